# baseline (speedup 1.0000x reference)
import functools

import jax
import jax.numpy as jnp
from jax import lax
from jax.experimental import pallas as pl
from jax.experimental.pallas import tpu as pltpu
from jax.experimental.pallas import tpu_sc as plsc

N = 100000
E = 1600000
H = 32
HH = 16
B = 128
G = 16

NC = 2
NS = 16

E_PAD = 1646592
CHUNKS = E_PAD // 128
TRASH = N
N_ACC = 100096
ROWS_PER_TILE = N_ACC // NS
ZROWS = 136
BLK = 4

RB = 5000
GRID = N // RB

_SC_PARAMS = pltpu.CompilerParams(use_tc_tiling_on_sc=False)


def _sc_mesh():
    return plsc.VectorSubcoreMesh(core_axis_name="c", subcore_axis_name="s")


def _zero_acc(acc, zbuf, s):
    def zb(i, carry):
        zbuf[i] = jnp.zeros((HH,), jnp.float32)
        return carry

    lax.fori_loop(0, ZROWS, zb, 0)
    base = s * ROWS_PER_TILE

    def zc(i, carry):
        pltpu.sync_copy(zbuf, acc.at[pl.ds(base + i * ZROWS, ZROWS)])
        return carry

    lax.fori_loop(0, ROWS_PER_TILE // ZROWS, zc, 0)
    return base


def _sc_agg(p, src_r, dst_r):
    ch_per_tile = CHUNKS // NS
    ntri = ch_per_tile // (3 * BLK)

    @functools.partial(
        pl.kernel,
        out_type=jax.ShapeDtypeStruct((NC, N_ACC, HH), jnp.float32),
        mesh=_sc_mesh(),
        scratch_types=[
            pltpu.VMEM_SHARED((N_ACC, HH), jnp.float32),
            pltpu.VMEM((ZROWS, HH), jnp.float32),
            pltpu.VMEM((BLK, 128), jnp.int32),
            pltpu.VMEM((BLK, 128), jnp.int32),
            pltpu.VMEM((BLK, 128), jnp.int32),
            pltpu.VMEM((BLK, 128), jnp.int32),
            pltpu.VMEM((BLK, 128), jnp.int32),
            pltpu.VMEM((BLK, 128), jnp.int32),
            pltpu.VMEM((BLK, 128, HH), jnp.float32),
            pltpu.VMEM((BLK, 128, HH), jnp.float32),
            pltpu.VMEM((BLK, 128, HH), jnp.float32),
            pltpu.SemaphoreType.DMA,
            pltpu.SemaphoreType.DMA,
            pltpu.SemaphoreType.DMA,
            pltpu.SemaphoreType.DMA,
            pltpu.SemaphoreType.DMA,
            pltpu.SemaphoreType.DMA,
        ],
        compiler_params=_SC_PARAMS,
    )
    def k(p_hbm, src_hbm, dst_hbm, out_hbm, acc, zbuf, sa, da, sb, db,
          sc_, dc_, rowsa, rowsb, rowsc, gsa, gsb, gsc, ssa, ssb, ssc):
        c = lax.axis_index("c")
        s = lax.axis_index("s")
        base = _zero_acc(acc, zbuf, s)
        plsc.subcore_barrier()

        ph = p_hbm.at[c]
        tile0 = s * ch_per_tile

        def load_fire(c0, sbuf, dbuf, rows, gsem):
            pltpu.sync_copy(src_hbm.at[pl.ds(c0, BLK)], sbuf)
            pltpu.sync_copy(dst_hbm.at[pl.ds(c0, BLK)], dbuf)
            for j in range(BLK):
                pltpu.async_copy(ph.at[sbuf.at[j]], rows.at[j], gsem)

        def drain_gather(sbuf, rows, gsem):
            for j in range(BLK):
                pltpu.make_async_copy(ph.at[sbuf.at[j]], rows.at[j],
                                      gsem).wait()

        def fire_scatter(dbuf, rows, ssem):
            for j in range(BLK):
                pltpu.async_copy(rows.at[j], acc.at[dbuf.at[j]], ssem,
                                 add=True)

        def drain_scatter(dbuf, rows, ssem):
            for j in range(BLK):
                pltpu.make_async_copy(rows.at[j], acc.at[dbuf.at[j]],
                                      ssem).wait()

        def body(i, carry):
            drain_gather(sa, rowsa, gsa)
            fire_scatter(da, rowsa, ssa)

            @pl.when(i > 0)
            def _():
                drain_scatter(dc_, rowsc, ssc)

            load_fire(tile0 + (3 * i + 2) * BLK, sc_, dc_, rowsc, gsc)
            drain_gather(sb, rowsb, gsb)
            fire_scatter(db, rowsb, ssb)
            drain_scatter(da, rowsa, ssa)

            @pl.when(i < ntri - 1)
            def _():
                load_fire(tile0 + (3 * i + 3) * BLK, sa, da, rowsa, gsa)

            drain_gather(sc_, rowsc, gsc)
            fire_scatter(dc_, rowsc, ssc)
            drain_scatter(db, rowsb, ssb)

            @pl.when(i < ntri - 1)
            def _():
                load_fire(tile0 + (3 * i + 4) * BLK, sb, db, rowsb, gsb)

            return carry

        load_fire(tile0, sa, da, rowsa, gsa)
        load_fire(tile0 + BLK, sb, db, rowsb, gsb)
        lax.fori_loop(0, ntri, body, 0)
        drain_scatter(dc_, rowsc, ssc)
        plsc.subcore_barrier()
        pltpu.sync_copy(acc.at[pl.ds(base, ROWS_PER_TILE)],
                        out_hbm.at[c].at[pl.ds(base, ROWS_PER_TILE)])

    return k(p, src_r, dst_r)


def _sc_deg(dst_r):
    dblk = 6
    ch_per_worker = CHUNKS // (NC * NS)
    nblk = ch_per_worker // dblk

    @functools.partial(
        pl.kernel,
        out_type=jax.ShapeDtypeStruct((NC, N_ACC, HH), jnp.float32),
        mesh=_sc_mesh(),
        scratch_types=[
            pltpu.VMEM_SHARED((N_ACC, HH), jnp.float32),
            pltpu.VMEM((ZROWS, HH), jnp.float32),
            pltpu.VMEM((dblk, 128), jnp.int32),
            pltpu.VMEM((128, HH), jnp.float32),
            pltpu.SemaphoreType.DMA,
        ],
        compiler_params=_SC_PARAMS,
    )
    def k(dst_hbm, out_hbm, acc, zbuf, dbuf, obuf, sem):
        c = lax.axis_index("c")
        s = lax.axis_index("s")

        def ob(i, carry):
            obuf[i] = jnp.ones((HH,), jnp.float32)
            return carry

        lax.fori_loop(0, 128, ob, 0)
        base = _zero_acc(acc, zbuf, s)
        plsc.subcore_barrier()

        wid = c * NS + s

        def body(b, carry):
            c0 = wid * ch_per_worker + b * dblk
            pltpu.sync_copy(dst_hbm.at[pl.ds(c0, dblk)], dbuf)
            for j in range(dblk):
                pltpu.async_copy(obuf, acc.at[dbuf.at[j]], sem, add=True)
            for j in range(dblk):
                pltpu.make_async_copy(obuf, acc.at[dbuf.at[j]], sem).wait()
            return carry

        lax.fori_loop(0, nblk, body, 0)
        plsc.subcore_barrier()
        pltpu.sync_copy(acc.at[pl.ds(base, ROWS_PER_TILE)],
                        out_hbm.at[c].at[pl.ds(base, ROWS_PER_TILE)])

    return k(dst_r)


def _dot(a, b):
    return jax.lax.dot_general(a, b, (((1,), (0,)), ((), ())),
                               precision=lax.Precision.HIGHEST,
                               preferred_element_type=jnp.float32)


def _tc_prep(cnt, x, w1):
    def body(cnt_ref, x_ref, w_ref, isq_ref, hw_ref, p_ref):
        deg = cnt_ref[0, :, 0:1] + cnt_ref[1, :, 0:1] + 1.0
        isq = lax.rsqrt(deg)
        isq_ref[...] = isq
        hw = _dot(x_ref[...], w_ref[...])
        hw_ref[...] = hw
        p = hw * isq
        p_ref[0] = p[:, :HH]
        p_ref[1] = p[:, HH:]

    return pl.pallas_call(
        body,
        grid=(GRID,),
        in_specs=[
            pl.BlockSpec((NC, RB, HH), lambda i: (0, i, 0)),
            pl.BlockSpec((RB, H), lambda i: (i, 0)),
            pl.BlockSpec((H, H), lambda i: (0, 0)),
        ],
        out_specs=[
            pl.BlockSpec((RB, 1), lambda i: (i, 0)),
            pl.BlockSpec((RB, H), lambda i: (i, 0)),
            pl.BlockSpec((NC, RB, HH), lambda i: (0, i, 0)),
        ],
        out_shape=[
            jax.ShapeDtypeStruct((N, 1), jnp.float32),
            jax.ShapeDtypeStruct((N, H), jnp.float32),
            jax.ShapeDtypeStruct((NC, N, HH), jnp.float32),
        ],
    )(cnt, x, w1)


def _tc_stats_apply(acc, hw, isq, b, g, be, wn):
    inv_n = 1.0 / N

    def body(acc_ref, hw_ref, isq_ref, b_ref, g_ref, be_ref, w_ref,
             mom_ref, hwn_ref, p_ref):
        phase = pl.program_id(0)
        isq = isq_ref[...]
        accc = jnp.concatenate([acc_ref[0], acc_ref[1]], axis=1)
        o = isq * accc + (isq * isq) * hw_ref[...] + b_ref[...]

        @pl.when(phase == 0)
        def _():
            @pl.when(pl.program_id(1) == 0)
            def _():
                mom_ref[...] = jnp.zeros_like(mom_ref)

            mom_ref[0:1, :] = mom_ref[0:1, :] + jnp.sum(o, 0, keepdims=True)
            mom_ref[1:2, :] = mom_ref[1:2, :] + jnp.sum(o * o, 0,
                                                        keepdims=True)

        @pl.when(phase == 1)
        def _():
            mean = mom_ref[0:1, :] * inv_n
            var = mom_ref[1:2, :] * inv_n - mean * mean
            inv = lax.rsqrt(var + 1e-5)
            h = jnp.maximum((o - mean) * inv * g_ref[...] + be_ref[...], 0.0)
            hwn = _dot(h, w_ref[...])
            hwn_ref[...] = hwn
            pn = hwn * isq
            p_ref[0] = pn[:, :HH]
            p_ref[1] = pn[:, HH:]

    mom, hwn, pn = pl.pallas_call(
        body,
        grid=(2, GRID),
        in_specs=[
            pl.BlockSpec((NC, RB, HH), lambda p_, i: (0, i, 0)),
            pl.BlockSpec((RB, H), lambda p_, i: (i, 0)),
            pl.BlockSpec((RB, 1), lambda p_, i: (i, 0)),
            pl.BlockSpec((1, H), lambda p_, i: (0, 0)),
            pl.BlockSpec((1, H), lambda p_, i: (0, 0)),
            pl.BlockSpec((1, H), lambda p_, i: (0, 0)),
            pl.BlockSpec((H, H), lambda p_, i: (0, 0)),
        ],
        out_specs=[
            pl.BlockSpec((8, H), lambda p_, i: (0, 0)),
            pl.BlockSpec((RB, H), lambda p_, i: (p_ * i, 0)),
            pl.BlockSpec((NC, RB, HH), lambda p_, i: (0, p_ * i, 0)),
        ],
        out_shape=[
            jax.ShapeDtypeStruct((8, H), jnp.float32),
            jax.ShapeDtypeStruct((N, H), jnp.float32),
            jax.ShapeDtypeStruct((NC, N, HH), jnp.float32),
        ],
    )(acc, hw, isq, b.reshape(1, H), g.reshape(1, H), be.reshape(1, H), wn)
    return hwn, pn


def _tc_pool_head(acc, hw, isq, b, batch2d, gf, gw, gb, pw1, pb1, pw2, pb2):
    t = pw2.shape[1]

    def body(acc_ref, hw_ref, isq_ref, b_ref, batch_ref, gf_ref, gw_ref,
             gb_ref, w1_ref, b1_ref, w2_ref, b2_ref, ps_ref, pc_ref, out_ref):
        i = pl.program_id(0)

        @pl.when(i == 0)
        def _():
            ps_ref[...] = jnp.zeros_like(ps_ref)
            pc_ref[...] = jnp.zeros_like(pc_ref)

        @pl.when(i < GRID)
        def _():
            isq = isq_ref[...]
            accc = jnp.concatenate([acc_ref[0], acc_ref[1]], axis=1)
            o = isq * accc + (isq * isq) * hw_ref[...] + b_ref[...]
            oneh = (batch_ref[...] == lax.broadcasted_iota(
                jnp.int32, (RB, B), 1)).astype(jnp.float32)
            ps = jax.lax.dot_general(oneh, o, (((0,), (0,)), ((), ())),
                                     precision=lax.Precision.HIGHEST,
                                     preferred_element_type=jnp.float32)
            pc = jax.lax.dot_general(oneh, jnp.ones((RB, 8), jnp.float32),
                                     (((0,), (0,)), ((), ())),
                                     precision=lax.Precision.HIGHEST,
                                     preferred_element_type=jnp.float32)
            ps_ref[...] = ps_ref[...] + ps
            pc_ref[...] = pc_ref[...] + pc

        @pl.when(i == GRID)
        def _():
            cnt = jnp.maximum(pc_ref[:, 0:1], 1.0)
            pooled = ps_ref[...] / cnt
            grepr = jnp.maximum(
                _dot(gf_ref[...], gw_ref[...]) + gb_ref[...], 0.0)
            comb = jnp.concatenate([pooled, grepr], axis=1)
            hid = jnp.maximum(_dot(comb, w1_ref[...]) + b1_ref[...], 0.0)
            out_ref[...] = _dot(hid, w2_ref[...]) + b2_ref[...]

    idx = lambda i: (jnp.minimum(i, GRID - 1), 0)
    idx3 = lambda i: (0, jnp.minimum(i, GRID - 1), 0)
    ps, pc, out = pl.pallas_call(
        body,
        grid=(GRID + 1,),
        in_specs=[
            pl.BlockSpec((NC, RB, HH), idx3),
            pl.BlockSpec((RB, H), idx),
            pl.BlockSpec((RB, 1), idx),
            pl.BlockSpec((1, H), lambda i: (0, 0)),
            pl.BlockSpec((RB, 1), idx),
            pl.BlockSpec((B, G), lambda i: (0, 0)),
            pl.BlockSpec((G, G), lambda i: (0, 0)),
            pl.BlockSpec((1, G), lambda i: (0, 0)),
            pl.BlockSpec((H + G, H), lambda i: (0, 0)),
            pl.BlockSpec((1, H), lambda i: (0, 0)),
            pl.BlockSpec((H, t), lambda i: (0, 0)),
            pl.BlockSpec((1, t), lambda i: (0, 0)),
        ],
        out_specs=[
            pl.BlockSpec((B, H), lambda i: (0, 0)),
            pl.BlockSpec((B, 8), lambda i: (0, 0)),
            pl.BlockSpec((B, t), lambda i: (0, 0)),
        ],
        out_shape=[
            jax.ShapeDtypeStruct((B, H), jnp.float32),
            jax.ShapeDtypeStruct((B, 8), jnp.float32),
            jax.ShapeDtypeStruct((B, t), jnp.float32),
        ],
    )(acc, hw, isq, b.reshape(1, H), batch2d, gf, gw, gb.reshape(1, G),
      pw1, pb1.reshape(1, H), pw2, pb2.reshape(1, t))
    return out


def kernel(x, edge_index, batch, global_features, W1, b1, W2, b2, W3, b3,
           g1, be1, g2, be2, gW, gb, pW1, pb1, pW2, pb2):
    src = edge_index[0]
    dst = edge_index[1]
    pad = E_PAD - E
    src_r = jnp.concatenate([src, jnp.zeros((pad,), jnp.int32)]).reshape(
        CHUNKS, 128)
    dst_r = jnp.concatenate([dst, jnp.full((pad,), TRASH, jnp.int32)]).reshape(
        CHUNKS, 128)
    batch2d = batch.reshape(N, 1)

    cnt = _sc_deg(dst_r)
    isq, hw1, p1 = _tc_prep(cnt, x, W1)
    acc1 = _sc_agg(p1, src_r, dst_r)
    hw2, p2 = _tc_stats_apply(acc1, hw1, isq, b1, g1, be1, W2)
    acc2 = _sc_agg(p2, src_r, dst_r)
    hw3, p3 = _tc_stats_apply(acc2, hw2, isq, b2, g2, be2, W3)
    acc3 = _sc_agg(p3, src_r, dst_r)
    return _tc_pool_head(acc3, hw3, isq, b3, batch2d, global_features,
                         gW, gb, pW1, pb1, pW2, pb2)


# speedup vs baseline: 1.0262x; 1.0262x over previous
import functools

import jax
import jax.numpy as jnp
from jax import lax
from jax.experimental import pallas as pl
from jax.experimental.pallas import tpu as pltpu
from jax.experimental.pallas import tpu_sc as plsc

N = 100000
E = 1600000
H = 32
HH = 16
B = 128
G = 16

NC = 2
NS = 16

E_PAD = 1638400
CHUNKS = E_PAD // 128
TRASH = N
N_ACC = 100096
ROWS_PER_TILE = N_ACC // NS
ZROWS = ROWS_PER_TILE // 16
BLK = 5

RB = 5000
GRID = N // RB

_SC_PARAMS = pltpu.CompilerParams(use_tc_tiling_on_sc=False)


def _sc_mesh():
    return plsc.VectorSubcoreMesh(core_axis_name="c", subcore_axis_name="s")


def _zero_acc(acc, zbuf, s):
    def zb(i, carry):
        zbuf[i] = jnp.zeros((HH,), jnp.float32)
        return carry

    lax.fori_loop(0, ZROWS, zb, 0)
    base = s * ROWS_PER_TILE
    for i in range(16):
        pltpu.sync_copy(zbuf, acc.at[pl.ds(base + i * ZROWS, ZROWS)])
    return base


def _sc_agg(p, src_r, dst_r):
    ch_per_tile = CHUNKS // NS
    npair = ch_per_tile // (2 * BLK)

    @functools.partial(
        pl.kernel,
        out_type=jax.ShapeDtypeStruct((NC, N_ACC, HH), jnp.float32),
        mesh=_sc_mesh(),
        scratch_types=[
            pltpu.VMEM_SHARED((N_ACC, HH), jnp.float32),
            pltpu.VMEM((ZROWS, HH), jnp.float32),
            pltpu.VMEM((BLK, 128), jnp.int32),
            pltpu.VMEM((BLK, 128), jnp.int32),
            pltpu.VMEM((BLK, 128), jnp.int32),
            pltpu.VMEM((BLK, 128), jnp.int32),
            pltpu.VMEM((BLK, 128, HH), jnp.float32),
            pltpu.VMEM((BLK, 128, HH), jnp.float32),
            pltpu.SemaphoreType.DMA,
            pltpu.SemaphoreType.DMA,
            pltpu.SemaphoreType.DMA,
            pltpu.SemaphoreType.DMA,
        ],
        compiler_params=_SC_PARAMS,
    )
    def k(p_hbm, src_hbm, dst_hbm, out_hbm, acc, zbuf, sa, da, sb, db,
          rowsa, rowsb, gsa, gsb, ssa, ssb):
        c = lax.axis_index("c")
        s = lax.axis_index("s")
        base = _zero_acc(acc, zbuf, s)
        plsc.subcore_barrier()

        ph = p_hbm.at[c]
        tile0 = s * ch_per_tile

        def load_fire(c0, sbuf, dbuf, rows, gsem):
            pltpu.sync_copy(src_hbm.at[pl.ds(c0, BLK)], sbuf)
            pltpu.sync_copy(dst_hbm.at[pl.ds(c0, BLK)], dbuf)
            for j in range(BLK):
                pltpu.async_copy(ph.at[sbuf.at[j]], rows.at[j], gsem)

        def drain_gather(sbuf, rows, gsem):
            for j in range(BLK):
                pltpu.make_async_copy(ph.at[sbuf.at[j]], rows.at[j],
                                      gsem).wait()

        def fire_scatter(dbuf, rows, ssem):
            for j in range(BLK):
                pltpu.async_copy(rows.at[j], acc.at[dbuf.at[j]], ssem,
                                 add=True)

        def drain_scatter(dbuf, rows, ssem):
            for j in range(BLK):
                pltpu.make_async_copy(rows.at[j], acc.at[dbuf.at[j]],
                                      ssem).wait()

        def body(i, carry):
            @pl.when(i > 0)
            def _():
                drain_scatter(db, rowsb, ssb)

            load_fire(tile0 + (2 * i + 1) * BLK, sb, db, rowsb, gsb)
            drain_gather(sa, rowsa, gsa)
            fire_scatter(da, rowsa, ssa)

            @pl.when(i < npair - 1)
            def _():
                drain_scatter(da, rowsa, ssa)
                load_fire(tile0 + (2 * i + 2) * BLK, sa, da, rowsa, gsa)

            drain_gather(sb, rowsb, gsb)
            fire_scatter(db, rowsb, ssb)
            return carry

        load_fire(tile0, sa, da, rowsa, gsa)
        lax.fori_loop(0, npair, body, 0)
        drain_scatter(da, rowsa, ssa)
        drain_scatter(db, rowsb, ssb)
        plsc.subcore_barrier()
        pltpu.sync_copy(acc.at[pl.ds(base, ROWS_PER_TILE)],
                        out_hbm.at[c].at[pl.ds(base, ROWS_PER_TILE)])

    return k(p, src_r, dst_r)


def _sc_deg(dst_r):
    dblk = 8
    ch_per_worker = CHUNKS // (NC * NS)
    nblk = ch_per_worker // dblk

    @functools.partial(
        pl.kernel,
        out_type=jax.ShapeDtypeStruct((NC, N_ACC, HH), jnp.float32),
        mesh=_sc_mesh(),
        scratch_types=[
            pltpu.VMEM_SHARED((N_ACC, HH), jnp.float32),
            pltpu.VMEM((ZROWS, HH), jnp.float32),
            pltpu.VMEM((dblk, 128), jnp.int32),
            pltpu.VMEM((128, HH), jnp.float32),
            pltpu.SemaphoreType.DMA,
        ],
        compiler_params=_SC_PARAMS,
    )
    def k(dst_hbm, out_hbm, acc, zbuf, dbuf, obuf, sem):
        c = lax.axis_index("c")
        s = lax.axis_index("s")

        def ob(i, carry):
            obuf[i] = jnp.ones((HH,), jnp.float32)
            return carry

        lax.fori_loop(0, 128, ob, 0)
        base = _zero_acc(acc, zbuf, s)
        plsc.subcore_barrier()

        wid = c * NS + s

        def body(b, carry):
            c0 = wid * ch_per_worker + b * dblk
            pltpu.sync_copy(dst_hbm.at[pl.ds(c0, dblk)], dbuf)
            for j in range(dblk):
                pltpu.async_copy(obuf, acc.at[dbuf.at[j]], sem, add=True)
            for j in range(dblk):
                pltpu.make_async_copy(obuf, acc.at[dbuf.at[j]], sem).wait()
            return carry

        lax.fori_loop(0, nblk, body, 0)
        plsc.subcore_barrier()
        pltpu.sync_copy(acc.at[pl.ds(base, ROWS_PER_TILE)],
                        out_hbm.at[c].at[pl.ds(base, ROWS_PER_TILE)])

    return k(dst_r)


def _dot(a, b):
    return jax.lax.dot_general(a, b, (((1,), (0,)), ((), ())),
                               precision=lax.Precision.HIGHEST,
                               preferred_element_type=jnp.float32)


def _tc_prep(cnt, x, w1):
    def body(cnt_ref, x_ref, w_ref, isq_ref, hw_ref, p_ref):
        deg = cnt_ref[0, :, 0:1] + cnt_ref[1, :, 0:1] + 1.0
        isq = lax.rsqrt(deg)
        isq_ref[...] = isq
        hw = _dot(x_ref[...], w_ref[...])
        hw_ref[...] = hw
        p = hw * isq
        p_ref[0] = p[:, :HH]
        p_ref[1] = p[:, HH:]

    return pl.pallas_call(
        body,
        grid=(GRID,),
        in_specs=[
            pl.BlockSpec((NC, RB, HH), lambda i: (0, i, 0)),
            pl.BlockSpec((RB, H), lambda i: (i, 0)),
            pl.BlockSpec((H, H), lambda i: (0, 0)),
        ],
        out_specs=[
            pl.BlockSpec((RB, 1), lambda i: (i, 0)),
            pl.BlockSpec((RB, H), lambda i: (i, 0)),
            pl.BlockSpec((NC, RB, HH), lambda i: (0, i, 0)),
        ],
        out_shape=[
            jax.ShapeDtypeStruct((N, 1), jnp.float32),
            jax.ShapeDtypeStruct((N, H), jnp.float32),
            jax.ShapeDtypeStruct((NC, N, HH), jnp.float32),
        ],
    )(cnt, x, w1)


def _tc_stats_apply(acc, hw, isq, b, g, be, wn):
    inv_n = 1.0 / N

    def body(acc_ref, hw_ref, isq_ref, b_ref, g_ref, be_ref, w_ref,
             mom_ref, hwn_ref, p_ref):
        phase = pl.program_id(0)
        isq = isq_ref[...]
        accc = jnp.concatenate([acc_ref[0], acc_ref[1]], axis=1)
        o = isq * accc + (isq * isq) * hw_ref[...] + b_ref[...]

        @pl.when(phase == 0)
        def _():
            @pl.when(pl.program_id(1) == 0)
            def _():
                mom_ref[...] = jnp.zeros_like(mom_ref)

            mom_ref[0:1, :] = mom_ref[0:1, :] + jnp.sum(o, 0, keepdims=True)
            mom_ref[1:2, :] = mom_ref[1:2, :] + jnp.sum(o * o, 0,
                                                        keepdims=True)

        @pl.when(phase == 1)
        def _():
            mean = mom_ref[0:1, :] * inv_n
            var = mom_ref[1:2, :] * inv_n - mean * mean
            inv = lax.rsqrt(var + 1e-5)
            h = jnp.maximum((o - mean) * inv * g_ref[...] + be_ref[...], 0.0)
            hwn = _dot(h, w_ref[...])
            hwn_ref[...] = hwn
            pn = hwn * isq
            p_ref[0] = pn[:, :HH]
            p_ref[1] = pn[:, HH:]

    mom, hwn, pn = pl.pallas_call(
        body,
        grid=(2, GRID),
        in_specs=[
            pl.BlockSpec((NC, RB, HH), lambda p_, i: (0, i, 0)),
            pl.BlockSpec((RB, H), lambda p_, i: (i, 0)),
            pl.BlockSpec((RB, 1), lambda p_, i: (i, 0)),
            pl.BlockSpec((1, H), lambda p_, i: (0, 0)),
            pl.BlockSpec((1, H), lambda p_, i: (0, 0)),
            pl.BlockSpec((1, H), lambda p_, i: (0, 0)),
            pl.BlockSpec((H, H), lambda p_, i: (0, 0)),
        ],
        out_specs=[
            pl.BlockSpec((8, H), lambda p_, i: (0, 0)),
            pl.BlockSpec((RB, H), lambda p_, i: (p_ * i, 0)),
            pl.BlockSpec((NC, RB, HH), lambda p_, i: (0, p_ * i, 0)),
        ],
        out_shape=[
            jax.ShapeDtypeStruct((8, H), jnp.float32),
            jax.ShapeDtypeStruct((N, H), jnp.float32),
            jax.ShapeDtypeStruct((NC, N, HH), jnp.float32),
        ],
    )(acc, hw, isq, b.reshape(1, H), g.reshape(1, H), be.reshape(1, H), wn)
    return hwn, pn


def _tc_pool_head(acc, hw, isq, b, batch2d, gf, gw, gb, pw1, pb1, pw2, pb2):
    t = pw2.shape[1]

    def body(acc_ref, hw_ref, isq_ref, b_ref, batch_ref, gf_ref, gw_ref,
             gb_ref, w1_ref, b1_ref, w2_ref, b2_ref, ps_ref, pc_ref, out_ref):
        i = pl.program_id(0)

        @pl.when(i == 0)
        def _():
            ps_ref[...] = jnp.zeros_like(ps_ref)
            pc_ref[...] = jnp.zeros_like(pc_ref)

        @pl.when(i < GRID)
        def _():
            isq = isq_ref[...]
            accc = jnp.concatenate([acc_ref[0], acc_ref[1]], axis=1)
            o = isq * accc + (isq * isq) * hw_ref[...] + b_ref[...]
            oneh = (batch_ref[...] == lax.broadcasted_iota(
                jnp.int32, (RB, B), 1)).astype(jnp.float32)
            ps = jax.lax.dot_general(oneh, o, (((0,), (0,)), ((), ())),
                                     precision=lax.Precision.HIGHEST,
                                     preferred_element_type=jnp.float32)
            pc = jax.lax.dot_general(oneh, jnp.ones((RB, 8), jnp.float32),
                                     (((0,), (0,)), ((), ())),
                                     precision=lax.Precision.HIGHEST,
                                     preferred_element_type=jnp.float32)
            ps_ref[...] = ps_ref[...] + ps
            pc_ref[...] = pc_ref[...] + pc

        @pl.when(i == GRID)
        def _():
            cnt = jnp.maximum(pc_ref[:, 0:1], 1.0)
            pooled = ps_ref[...] / cnt
            grepr = jnp.maximum(
                _dot(gf_ref[...], gw_ref[...]) + gb_ref[...], 0.0)
            comb = jnp.concatenate([pooled, grepr], axis=1)
            hid = jnp.maximum(_dot(comb, w1_ref[...]) + b1_ref[...], 0.0)
            out_ref[...] = _dot(hid, w2_ref[...]) + b2_ref[...]

    idx = lambda i: (jnp.minimum(i, GRID - 1), 0)
    idx3 = lambda i: (0, jnp.minimum(i, GRID - 1), 0)
    ps, pc, out = pl.pallas_call(
        body,
        grid=(GRID + 1,),
        in_specs=[
            pl.BlockSpec((NC, RB, HH), idx3),
            pl.BlockSpec((RB, H), idx),
            pl.BlockSpec((RB, 1), idx),
            pl.BlockSpec((1, H), lambda i: (0, 0)),
            pl.BlockSpec((RB, 1), idx),
            pl.BlockSpec((B, G), lambda i: (0, 0)),
            pl.BlockSpec((G, G), lambda i: (0, 0)),
            pl.BlockSpec((1, G), lambda i: (0, 0)),
            pl.BlockSpec((H + G, H), lambda i: (0, 0)),
            pl.BlockSpec((1, H), lambda i: (0, 0)),
            pl.BlockSpec((H, t), lambda i: (0, 0)),
            pl.BlockSpec((1, t), lambda i: (0, 0)),
        ],
        out_specs=[
            pl.BlockSpec((B, H), lambda i: (0, 0)),
            pl.BlockSpec((B, 8), lambda i: (0, 0)),
            pl.BlockSpec((B, t), lambda i: (0, 0)),
        ],
        out_shape=[
            jax.ShapeDtypeStruct((B, H), jnp.float32),
            jax.ShapeDtypeStruct((B, 8), jnp.float32),
            jax.ShapeDtypeStruct((B, t), jnp.float32),
        ],
    )(acc, hw, isq, b.reshape(1, H), batch2d, gf, gw, gb.reshape(1, G),
      pw1, pb1.reshape(1, H), pw2, pb2.reshape(1, t))
    return out


def kernel(x, edge_index, batch, global_features, W1, b1, W2, b2, W3, b3,
           g1, be1, g2, be2, gW, gb, pW1, pb1, pW2, pb2):
    src = edge_index[0]
    dst = edge_index[1]
    pad = E_PAD - E
    src_r = jnp.concatenate([src, jnp.zeros((pad,), jnp.int32)]).reshape(
        CHUNKS, 128)
    dst_r = jnp.concatenate([dst, jnp.full((pad,), TRASH, jnp.int32)]).reshape(
        CHUNKS, 128)
    batch2d = batch.reshape(N, 1)

    cnt = _sc_deg(dst_r)
    isq, hw1, p1 = _tc_prep(cnt, x, W1)
    acc1 = _sc_agg(p1, src_r, dst_r)
    hw2, p2 = _tc_stats_apply(acc1, hw1, isq, b1, g1, be1, W2)
    acc2 = _sc_agg(p2, src_r, dst_r)
    hw3, p3 = _tc_stats_apply(acc2, hw2, isq, b2, g2, be2, W3)
    acc3 = _sc_agg(p3, src_r, dst_r)
    return _tc_pool_head(acc3, hw3, isq, b3, batch2d, global_features,
                         gW, gb, pW1, pb1, pW2, pb2)


# speedup vs baseline: 1.0313x; 1.0050x over previous
import functools

import jax
import jax.numpy as jnp
from jax import lax
from jax.experimental import pallas as pl
from jax.experimental.pallas import tpu as pltpu
from jax.experimental.pallas import tpu_sc as plsc

N = 100000
E = 1600000
H = 32
HH = 16
B = 128
G = 16

NC = 2
NS = 16

E_PAD = 1638400
CHUNKS = E_PAD // 128
TRASH = N
N_ACC = 100096
ROWS_PER_TILE = N_ACC // NS
ZROWS = ROWS_PER_TILE // 16
BLK = 5

RB = 5000
GRID = N // RB

_SC_PARAMS = pltpu.CompilerParams(use_tc_tiling_on_sc=False)


def _sc_mesh():
    return plsc.VectorSubcoreMesh(core_axis_name="c", subcore_axis_name="s")


def _zero_acc(acc, zbuf, s):
    def zb(i, carry):
        zbuf[i] = jnp.zeros((HH,), jnp.float32)
        return carry

    lax.fori_loop(0, ZROWS, zb, 0)
    base = s * ROWS_PER_TILE
    for i in range(16):
        pltpu.sync_copy(zbuf, acc.at[pl.ds(base + i * ZROWS, ZROWS)])
    return base


def _sc_agg(p, src_r, dst_r):
    ch_per_tile = CHUNKS // NS
    npair = ch_per_tile // (2 * BLK)

    @functools.partial(
        pl.kernel,
        out_type=jax.ShapeDtypeStruct((NC, N_ACC, HH), jnp.float32),
        mesh=_sc_mesh(),
        scratch_types=[
            pltpu.VMEM_SHARED((N_ACC, HH), jnp.float32),
            pltpu.VMEM((ZROWS, HH), jnp.float32),
            pltpu.VMEM((BLK, 128), jnp.int32),
            pltpu.VMEM((BLK, 128), jnp.int32),
            pltpu.VMEM((BLK, 128), jnp.int32),
            pltpu.VMEM((BLK, 128), jnp.int32),
            pltpu.VMEM((BLK, 128, HH), jnp.float32),
            pltpu.VMEM((BLK, 128, HH), jnp.float32),
            pltpu.SemaphoreType.DMA,
            pltpu.SemaphoreType.DMA,
            pltpu.SemaphoreType.DMA,
            pltpu.SemaphoreType.DMA,
        ],
        compiler_params=_SC_PARAMS,
    )
    def k(p_hbm, src_hbm, dst_hbm, out_hbm, acc, zbuf, sa, da, sb, db,
          rowsa, rowsb, gsa, gsb, ssa, ssb):
        c = lax.axis_index("c")
        s = lax.axis_index("s")
        base = _zero_acc(acc, zbuf, s)
        plsc.subcore_barrier()

        ph = p_hbm.at[c]
        tile0 = s * ch_per_tile

        def load_fire(c0, sbuf, dbuf, rows, gsem):
            pltpu.sync_copy(src_hbm.at[pl.ds(c0, BLK)], sbuf)
            pltpu.sync_copy(dst_hbm.at[pl.ds(c0, BLK)], dbuf)
            for j in range(BLK):
                pltpu.async_copy(ph.at[sbuf.at[j]], rows.at[j], gsem)

        def drain_gather(sbuf, rows, gsem):
            for j in range(BLK):
                pltpu.make_async_copy(ph.at[sbuf.at[j]], rows.at[j],
                                      gsem).wait()

        def fire_scatter(dbuf, rows, ssem):
            for j in range(BLK):
                pltpu.async_copy(rows.at[j], acc.at[dbuf.at[j]], ssem,
                                 add=True)

        def drain_scatter(dbuf, rows, ssem):
            for j in range(BLK):
                pltpu.make_async_copy(rows.at[j], acc.at[dbuf.at[j]],
                                      ssem).wait()

        def body(i, carry):
            @pl.when(i > 0)
            def _():
                drain_scatter(db, rowsb, ssb)

            load_fire(tile0 + (2 * i + 1) * BLK, sb, db, rowsb, gsb)
            drain_gather(sa, rowsa, gsa)
            fire_scatter(da, rowsa, ssa)

            @pl.when(i < npair - 1)
            def _():
                drain_scatter(da, rowsa, ssa)
                load_fire(tile0 + (2 * i + 2) * BLK, sa, da, rowsa, gsa)

            drain_gather(sb, rowsb, gsb)
            fire_scatter(db, rowsb, ssb)
            return carry

        load_fire(tile0, sa, da, rowsa, gsa)
        lax.fori_loop(0, npair, body, 0)
        drain_scatter(da, rowsa, ssa)
        drain_scatter(db, rowsb, ssb)
        plsc.subcore_barrier()
        pltpu.sync_copy(acc.at[pl.ds(base, ROWS_PER_TILE)],
                        out_hbm.at[c].at[pl.ds(base, ROWS_PER_TILE)])

    return k(p, src_r, dst_r)


def _sc_deg(dst_r):
    dblk = 8
    ch_per_worker = CHUNKS // (NC * NS)
    nblk = ch_per_worker // dblk

    @functools.partial(
        pl.kernel,
        out_type=jax.ShapeDtypeStruct((NC, N_ACC, HH), jnp.float32),
        mesh=_sc_mesh(),
        scratch_types=[
            pltpu.VMEM_SHARED((N_ACC, HH), jnp.float32),
            pltpu.VMEM((ZROWS, HH), jnp.float32),
            pltpu.VMEM((dblk, 128), jnp.int32),
            pltpu.VMEM((dblk, 128), jnp.int32),
            pltpu.VMEM((128, HH), jnp.float32),
            pltpu.SemaphoreType.DMA,
            pltpu.SemaphoreType.DMA,
        ],
        compiler_params=_SC_PARAMS,
    )
    def k(dst_hbm, out_hbm, acc, zbuf, dbufa, dbufb, obuf, sema, semb):
        c = lax.axis_index("c")
        s = lax.axis_index("s")

        def ob(i, carry):
            obuf[i] = jnp.ones((HH,), jnp.float32)
            return carry

        lax.fori_loop(0, 128, ob, 0)
        base = _zero_acc(acc, zbuf, s)
        plsc.subcore_barrier()

        wid = c * NS + s
        base0 = wid * ch_per_worker
        npair = nblk // 2

        def load(c0, dbuf):
            pltpu.sync_copy(dst_hbm.at[pl.ds(c0, dblk)], dbuf)

        def fire(dbuf, sem):
            for j in range(dblk):
                pltpu.async_copy(obuf, acc.at[dbuf.at[j]], sem, add=True)

        def drain(dbuf, sem):
            for j in range(dblk):
                pltpu.make_async_copy(obuf, acc.at[dbuf.at[j]], sem).wait()

        def body(i, carry):
            load(base0 + (2 * i + 1) * dblk, dbufb)
            fire(dbufb, semb)
            drain(dbufa, sema)

            @pl.when(i < npair - 1)
            def _():
                load(base0 + (2 * i + 2) * dblk, dbufa)
                fire(dbufa, sema)

            drain(dbufb, semb)
            return carry

        load(base0, dbufa)
        fire(dbufa, sema)
        lax.fori_loop(0, npair, body, 0)
        plsc.subcore_barrier()
        pltpu.sync_copy(acc.at[pl.ds(base, ROWS_PER_TILE)],
                        out_hbm.at[c].at[pl.ds(base, ROWS_PER_TILE)])

    return k(dst_r)


def _dot(a, b):
    return jax.lax.dot_general(a, b, (((1,), (0,)), ((), ())),
                               precision=lax.Precision.HIGHEST,
                               preferred_element_type=jnp.float32)


def _tc_prep(cnt, x, w1):
    def body(cnt_ref, x_ref, w_ref, isq_ref, hw_ref, p_ref):
        deg = cnt_ref[0, :, 0:1] + cnt_ref[1, :, 0:1] + 1.0
        isq = lax.rsqrt(deg)
        isq_ref[...] = isq
        hw = _dot(x_ref[...], w_ref[...])
        hw_ref[...] = hw
        p = hw * isq
        p_ref[0] = p[:, :HH]
        p_ref[1] = p[:, HH:]

    return pl.pallas_call(
        body,
        grid=(GRID,),
        in_specs=[
            pl.BlockSpec((NC, RB, HH), lambda i: (0, i, 0)),
            pl.BlockSpec((RB, H), lambda i: (i, 0)),
            pl.BlockSpec((H, H), lambda i: (0, 0)),
        ],
        out_specs=[
            pl.BlockSpec((RB, 1), lambda i: (i, 0)),
            pl.BlockSpec((RB, H), lambda i: (i, 0)),
            pl.BlockSpec((NC, RB, HH), lambda i: (0, i, 0)),
        ],
        out_shape=[
            jax.ShapeDtypeStruct((N, 1), jnp.float32),
            jax.ShapeDtypeStruct((N, H), jnp.float32),
            jax.ShapeDtypeStruct((NC, N, HH), jnp.float32),
        ],
    )(cnt, x, w1)


def _tc_stats_apply(acc, hw, isq, b, g, be, wn):
    inv_n = 1.0 / N

    def body(acc_ref, hw_ref, isq_ref, b_ref, g_ref, be_ref, w_ref,
             mom_ref, hwn_ref, p_ref):
        phase = pl.program_id(0)
        isq = isq_ref[...]
        accc = jnp.concatenate([acc_ref[0], acc_ref[1]], axis=1)
        o = isq * accc + (isq * isq) * hw_ref[...] + b_ref[...]

        @pl.when(phase == 0)
        def _():
            @pl.when(pl.program_id(1) == 0)
            def _():
                mom_ref[...] = jnp.zeros_like(mom_ref)

            mom_ref[0:1, :] = mom_ref[0:1, :] + jnp.sum(o, 0, keepdims=True)
            mom_ref[1:2, :] = mom_ref[1:2, :] + jnp.sum(o * o, 0,
                                                        keepdims=True)

        @pl.when(phase == 1)
        def _():
            mean = mom_ref[0:1, :] * inv_n
            var = mom_ref[1:2, :] * inv_n - mean * mean
            inv = lax.rsqrt(var + 1e-5)
            h = jnp.maximum((o - mean) * inv * g_ref[...] + be_ref[...], 0.0)
            hwn = _dot(h, w_ref[...])
            hwn_ref[...] = hwn
            pn = hwn * isq
            p_ref[0] = pn[:, :HH]
            p_ref[1] = pn[:, HH:]

    mom, hwn, pn = pl.pallas_call(
        body,
        grid=(2, GRID),
        in_specs=[
            pl.BlockSpec((NC, RB, HH), lambda p_, i: (0, i, 0)),
            pl.BlockSpec((RB, H), lambda p_, i: (i, 0)),
            pl.BlockSpec((RB, 1), lambda p_, i: (i, 0)),
            pl.BlockSpec((1, H), lambda p_, i: (0, 0)),
            pl.BlockSpec((1, H), lambda p_, i: (0, 0)),
            pl.BlockSpec((1, H), lambda p_, i: (0, 0)),
            pl.BlockSpec((H, H), lambda p_, i: (0, 0)),
        ],
        out_specs=[
            pl.BlockSpec((8, H), lambda p_, i: (0, 0)),
            pl.BlockSpec((RB, H), lambda p_, i: (p_ * i, 0)),
            pl.BlockSpec((NC, RB, HH), lambda p_, i: (0, p_ * i, 0)),
        ],
        out_shape=[
            jax.ShapeDtypeStruct((8, H), jnp.float32),
            jax.ShapeDtypeStruct((N, H), jnp.float32),
            jax.ShapeDtypeStruct((NC, N, HH), jnp.float32),
        ],
    )(acc, hw, isq, b.reshape(1, H), g.reshape(1, H), be.reshape(1, H), wn)
    return hwn, pn


def _tc_pool_head(acc, hw, isq, b, batch2d, gf, gw, gb, pw1, pb1, pw2, pb2):
    t = pw2.shape[1]

    def body(acc_ref, hw_ref, isq_ref, b_ref, batch_ref, gf_ref, gw_ref,
             gb_ref, w1_ref, b1_ref, w2_ref, b2_ref, ps_ref, pc_ref, out_ref):
        i = pl.program_id(0)

        @pl.when(i == 0)
        def _():
            ps_ref[...] = jnp.zeros_like(ps_ref)
            pc_ref[...] = jnp.zeros_like(pc_ref)

        @pl.when(i < GRID)
        def _():
            isq = isq_ref[...]
            accc = jnp.concatenate([acc_ref[0], acc_ref[1]], axis=1)
            o = isq * accc + (isq * isq) * hw_ref[...] + b_ref[...]
            oneh = (batch_ref[...] == lax.broadcasted_iota(
                jnp.int32, (RB, B), 1)).astype(jnp.float32)
            ps = jax.lax.dot_general(oneh, o, (((0,), (0,)), ((), ())),
                                     precision=lax.Precision.HIGHEST,
                                     preferred_element_type=jnp.float32)
            pc = jax.lax.dot_general(oneh, jnp.ones((RB, 8), jnp.float32),
                                     (((0,), (0,)), ((), ())),
                                     precision=lax.Precision.HIGHEST,
                                     preferred_element_type=jnp.float32)
            ps_ref[...] = ps_ref[...] + ps
            pc_ref[...] = pc_ref[...] + pc

        @pl.when(i == GRID)
        def _():
            cnt = jnp.maximum(pc_ref[:, 0:1], 1.0)
            pooled = ps_ref[...] / cnt
            grepr = jnp.maximum(
                _dot(gf_ref[...], gw_ref[...]) + gb_ref[...], 0.0)
            comb = jnp.concatenate([pooled, grepr], axis=1)
            hid = jnp.maximum(_dot(comb, w1_ref[...]) + b1_ref[...], 0.0)
            out_ref[...] = _dot(hid, w2_ref[...]) + b2_ref[...]

    idx = lambda i: (jnp.minimum(i, GRID - 1), 0)
    idx3 = lambda i: (0, jnp.minimum(i, GRID - 1), 0)
    ps, pc, out = pl.pallas_call(
        body,
        grid=(GRID + 1,),
        in_specs=[
            pl.BlockSpec((NC, RB, HH), idx3),
            pl.BlockSpec((RB, H), idx),
            pl.BlockSpec((RB, 1), idx),
            pl.BlockSpec((1, H), lambda i: (0, 0)),
            pl.BlockSpec((RB, 1), idx),
            pl.BlockSpec((B, G), lambda i: (0, 0)),
            pl.BlockSpec((G, G), lambda i: (0, 0)),
            pl.BlockSpec((1, G), lambda i: (0, 0)),
            pl.BlockSpec((H + G, H), lambda i: (0, 0)),
            pl.BlockSpec((1, H), lambda i: (0, 0)),
            pl.BlockSpec((H, t), lambda i: (0, 0)),
            pl.BlockSpec((1, t), lambda i: (0, 0)),
        ],
        out_specs=[
            pl.BlockSpec((B, H), lambda i: (0, 0)),
            pl.BlockSpec((B, 8), lambda i: (0, 0)),
            pl.BlockSpec((B, t), lambda i: (0, 0)),
        ],
        out_shape=[
            jax.ShapeDtypeStruct((B, H), jnp.float32),
            jax.ShapeDtypeStruct((B, 8), jnp.float32),
            jax.ShapeDtypeStruct((B, t), jnp.float32),
        ],
    )(acc, hw, isq, b.reshape(1, H), batch2d, gf, gw, gb.reshape(1, G),
      pw1, pb1.reshape(1, H), pw2, pb2.reshape(1, t))
    return out


def kernel(x, edge_index, batch, global_features, W1, b1, W2, b2, W3, b3,
           g1, be1, g2, be2, gW, gb, pW1, pb1, pW2, pb2):
    src = edge_index[0]
    dst = edge_index[1]
    pad = E_PAD - E
    src_r = jnp.concatenate([src, jnp.zeros((pad,), jnp.int32)]).reshape(
        CHUNKS, 128)
    dst_r = jnp.concatenate([dst, jnp.full((pad,), TRASH, jnp.int32)]).reshape(
        CHUNKS, 128)
    batch2d = batch.reshape(N, 1)

    cnt = _sc_deg(dst_r)
    isq, hw1, p1 = _tc_prep(cnt, x, W1)
    acc1 = _sc_agg(p1, src_r, dst_r)
    hw2, p2 = _tc_stats_apply(acc1, hw1, isq, b1, g1, be1, W2)
    acc2 = _sc_agg(p2, src_r, dst_r)
    hw3, p3 = _tc_stats_apply(acc2, hw2, isq, b2, g2, be2, W3)
    acc3 = _sc_agg(p3, src_r, dst_r)
    return _tc_pool_head(acc3, hw3, isq, b3, batch2d, global_features,
                         gW, gb, pW1, pb1, pW2, pb2)
